# Initial kernel scaffold; baseline (speedup 1.0000x reference)
import jax
import jax.numpy as jnp
from jax.experimental import pallas as pl


def kernel(x, seq_idx, seq_mask, node_type, gamma, beta):
    raise NotImplementedError("write your pallas kernel here")


# speedup vs baseline: 3.6013x; 3.6013x over previous
import functools

import jax
import jax.numpy as jnp
from jax import lax
from jax.experimental import pallas as pl
from jax.experimental.pallas import tpu as pltpu
from jax.experimental.pallas import tpu_sc as plsc

B, N, C, S = 8, 16384, 128, 8192
LN_EPS = 1e-5

@functools.cache
def _build_flag_kernel():
    info = plsc.get_sparse_core_info()
    NC, NS, L = info.num_cores, info.num_subcores, info.num_lanes
    NW = NC * NS
    WPB = NW // B
    CHUNK = N // WPB

    mesh = plsc.VectorSubcoreMesh(core_axis_name="c", subcore_axis_name="s")

    @functools.partial(
        pl.kernel,
        mesh=mesh,
        out_type=jax.ShapeDtypeStruct((B * N,), jnp.float32),
        scratch_types=[
            pltpu.VMEM((S,), jnp.int32),
            pltpu.VMEM((S,), jnp.int32),
            pltpu.VMEM((CHUNK,), jnp.float32),
        ],
        compiler_params=pltpu.CompilerParams(needs_layout_passes=False),
    )
    def flag_kernel(idx_hbm, msk_hbm, flags_hbm, idx_v, msk_v, flg_v):
        wid = lax.axis_index("s") * NC + lax.axis_index("c")
        b = wid // WPB
        base = (wid % WPB) * CHUNK

        zeros16 = jnp.zeros((L,), jnp.float32)

        def zero_body(i, carry):
            flg_v[pl.ds(i * L, L)] = zeros16
            return carry

        lax.fori_loop(0, CHUNK // L, zero_body, 0)

        pltpu.sync_copy(idx_hbm.at[pl.ds(b * S, S)], idx_v)
        pltpu.sync_copy(msk_hbm.at[pl.ds(b * S, S)], msk_v)

        ones16 = jnp.ones((L,), jnp.float32)

        def scatter_body(i, carry):
            ii = idx_v[pl.ds(i * L, L)]
            mm = msk_v[pl.ds(i * L, L)]
            ii = jnp.clip(ii, 0, N - 1)
            rel = ii - base
            ok = (mm != 0) & (rel >= 0) & (rel < CHUNK)
            rel = jnp.clip(rel, 0, CHUNK - 1)
            plsc.store_scatter(flg_v, [rel], ones16, mask=ok)
            return carry

        lax.fori_loop(0, S // L, scatter_body, 0)

        pltpu.sync_copy(flg_v, flags_hbm.at[pl.ds(b * N + base, CHUNK)])

    return flag_kernel


ROW_TILE = 1024


def _ln_body(x_ref, f_ref, g_ref, bt_ref, o_ref):
    xv = x_ref[0]
    fv = f_ref[0]
    mu = jnp.mean(xv, axis=-1, keepdims=True)
    xc = xv - mu
    var = jnp.mean(xc * xc, axis=-1, keepdims=True)
    ln = xc * lax.rsqrt(var + LN_EPS) * g_ref[...] + bt_ref[...]
    o_ref[0] = xv + jnp.where(fv > 0.0, ln, 0.0)


def kernel(x, seq_idx, seq_mask, node_type, gamma, beta):
    del node_type
    idx_flat = seq_idx.astype(jnp.int32).reshape(-1)
    msk_flat = seq_mask.astype(jnp.int32).reshape(-1)

    flags = _build_flag_kernel()(idx_flat, msk_flat)
    flags3 = flags.reshape(B, N, 1)

    out = pl.pallas_call(
        _ln_body,
        grid=(B, N // ROW_TILE),
        in_specs=[
            pl.BlockSpec((1, ROW_TILE, C), lambda b, j: (b, j, 0)),
            pl.BlockSpec((1, ROW_TILE, 1), lambda b, j: (b, j, 0)),
            pl.BlockSpec((1, C), lambda b, j: (0, 0)),
            pl.BlockSpec((1, C), lambda b, j: (0, 0)),
        ],
        out_specs=pl.BlockSpec((1, ROW_TILE, C), lambda b, j: (b, j, 0)),
        out_shape=jax.ShapeDtypeStruct((B, N, C), jnp.float32),
    )(x, flags3, gamma.reshape(1, C), beta.reshape(1, C))
    return out


# speedup vs baseline: 5.5368x; 1.5374x over previous
import functools

import jax
import jax.numpy as jnp
from jax import lax
from jax.experimental import pallas as pl
from jax.experimental.pallas import tpu as pltpu
from jax.experimental.pallas import tpu_sc as plsc

B, N, C, S = 8, 16384, 128, 8192
LN_EPS = 1e-5
LANES = 128
NROW = N // LANES


@functools.cache
def _build_flag_kernel():
    info = plsc.get_sparse_core_info()
    NC, NS, L = info.num_cores, info.num_subcores, info.num_lanes
    NW = NC * NS
    WPB = NW // B
    CHUNK = N // WPB
    CROWS = CHUNK // LANES

    mesh = plsc.VectorSubcoreMesh(core_axis_name="c", subcore_axis_name="s")

    @functools.partial(
        pl.kernel,
        mesh=mesh,
        out_type=jax.ShapeDtypeStruct((B, NROW, LANES), jnp.float32),
        scratch_types=[
            pltpu.VMEM((S,), jnp.int32),
            pltpu.VMEM((S,), jnp.int32),
            pltpu.VMEM((CROWS, LANES), jnp.float32),
        ],
        compiler_params=pltpu.CompilerParams(needs_layout_passes=False),
    )
    def flag_kernel(idx_hbm, msk_hbm, flags_hbm, idx_v, msk_v, flg_v):
        wid = lax.axis_index("s") * NC + lax.axis_index("c")
        b = wid // WPB
        base = (wid % WPB) * CHUNK

        zeros16 = jnp.zeros((L,), jnp.float32)

        def zero_body(i, carry):
            flg_v[i // (LANES // L), pl.ds((i % (LANES // L)) * L, L)] = zeros16
            return carry

        lax.fori_loop(0, CHUNK // L, zero_body, 0)

        pltpu.sync_copy(idx_hbm.at[pl.ds(b * S, S)], idx_v)
        pltpu.sync_copy(msk_hbm.at[pl.ds(b * S, S)], msk_v)

        ones16 = jnp.ones((L,), jnp.float32)

        def scatter_body(i, carry):
            ii = idx_v[pl.ds(i * L, L)]
            mm = msk_v[pl.ds(i * L, L)]
            ii = jnp.clip(ii, 0, N - 1)
            rel = ii - base
            ok = (mm != 0) & (rel >= 0) & (rel < CHUNK)
            rel = jnp.clip(rel, 0, CHUNK - 1)
            plsc.store_scatter(
                flg_v,
                [lax.shift_right_logical(rel, 7), rel & (LANES - 1)],
                ones16,
                mask=ok,
            )
            return carry

        lax.fori_loop(0, S // L, scatter_body, 0)

        pltpu.sync_copy(flg_v, flags_hbm.at[b, pl.ds((wid % WPB) * CROWS, CROWS)])

    return flag_kernel


ROW_TILE = 1024
FROWS = ROW_TILE // LANES


def _ln_body(x_ref, f_ref, g_ref, bt_ref, o_ref):
    xv = x_ref[0]
    fv = f_ref[0]
    grep = jnp.broadcast_to(fv[:, None, :], (FROWS, LANES, LANES))
    grep = grep.reshape(ROW_TILE, LANES)
    sub = lax.broadcasted_iota(jnp.int32, (ROW_TILE, LANES), 0)
    lane = lax.broadcasted_iota(jnp.int32, (ROW_TILE, LANES), 1)
    sel = jnp.where(lane == (sub & (LANES - 1)), grep, 0.0)
    colflag = jnp.sum(sel, axis=-1, keepdims=True)

    mu = jnp.mean(xv, axis=-1, keepdims=True)
    xc = xv - mu
    var = jnp.mean(xc * xc, axis=-1, keepdims=True)
    ln = xc * lax.rsqrt(var + LN_EPS) * g_ref[...] + bt_ref[...]
    o_ref[0] = xv + ln * colflag


def kernel(x, seq_idx, seq_mask, node_type, gamma, beta):
    del node_type
    idx_flat = seq_idx.astype(jnp.int32).reshape(-1)
    msk_flat = seq_mask.astype(jnp.int32).reshape(-1)

    flags = _build_flag_kernel()(idx_flat, msk_flat)

    out = pl.pallas_call(
        _ln_body,
        grid=(B, N // ROW_TILE),
        in_specs=[
            pl.BlockSpec((1, ROW_TILE, C), lambda b, j: (b, j, 0)),
            pl.BlockSpec((1, FROWS, LANES), lambda b, j: (b, j, 0)),
            pl.BlockSpec((1, C), lambda b, j: (0, 0)),
            pl.BlockSpec((1, C), lambda b, j: (0, 0)),
        ],
        out_specs=pl.BlockSpec((1, ROW_TILE, C), lambda b, j: (b, j, 0)),
        out_shape=jax.ShapeDtypeStruct((B, N, C), jnp.float32),
    )(x, flags, gamma.reshape(1, C), beta.reshape(1, C))
    return out


# speedup vs baseline: 8.1953x; 1.4802x over previous
import functools

import jax
import jax.numpy as jnp
from jax import lax
from jax.experimental import pallas as pl
from jax.experimental.pallas import tpu as pltpu
from jax.experimental.pallas import tpu_sc as plsc

B, N, C, S = 8, 16384, 128, 8192
LN_EPS = 1e-5
LANES = 128
NROW = N // LANES


@functools.cache
def _build_flag_kernel():
    info = plsc.get_sparse_core_info()
    NC, NS, L = info.num_cores, info.num_subcores, info.num_lanes
    NW = NC * NS
    WPB = NW // B
    CHUNK = N // WPB
    CROWS = CHUNK // LANES

    mesh = plsc.VectorSubcoreMesh(core_axis_name="c", subcore_axis_name="s")

    @functools.partial(
        pl.kernel,
        mesh=mesh,
        out_type=jax.ShapeDtypeStruct((B, NROW, LANES), jnp.float32),
        scratch_types=[
            pltpu.VMEM((S,), jnp.int32),
            pltpu.VMEM((S,), jnp.int32),
            pltpu.VMEM((CROWS, LANES), jnp.float32),
        ],
        compiler_params=pltpu.CompilerParams(needs_layout_passes=False),
    )
    def flag_kernel(idx_hbm, msk_hbm, flags_hbm, idx_v, msk_v, flg_v):
        wid = lax.axis_index("s") * NC + lax.axis_index("c")
        b = wid // WPB
        base = (wid % WPB) * CHUNK

        zeros16 = jnp.zeros((L,), jnp.float32)

        def zero_body(i, carry):
            flg_v[i // (LANES // L), pl.ds((i % (LANES // L)) * L, L)] = zeros16
            return carry

        lax.fori_loop(0, CHUNK // L, zero_body, 0)

        pltpu.sync_copy(idx_hbm.at[pl.ds(b * S, S)], idx_v)
        pltpu.sync_copy(msk_hbm.at[pl.ds(b * S, S)], msk_v)

        ones16 = jnp.ones((L,), jnp.float32)

        def scatter_body(i, carry):
            ii = idx_v[pl.ds(i * L, L)]
            mm = msk_v[pl.ds(i * L, L)]
            ii = jnp.clip(ii, 0, N - 1)
            rel = ii - base
            ok = (mm != 0) & (rel >= 0) & (rel < CHUNK)
            rel = jnp.clip(rel, 0, CHUNK - 1)
            plsc.store_scatter(
                flg_v,
                [lax.shift_right_logical(rel, 7), rel & (LANES - 1)],
                ones16,
                mask=ok,
            )
            return carry

        lax.fori_loop(0, S // L, scatter_body, 0)

        pltpu.sync_copy(flg_v, flags_hbm.at[b, pl.ds((wid % WPB) * CROWS, CROWS)])

    return flag_kernel


ROW_TILE = 4096
FROWS = ROW_TILE // LANES


def _ln_body(x_ref, f_ref, g_ref, bt_ref, o_ref):
    xv = x_ref[0]
    fv = f_ref[0]
    grep = jnp.broadcast_to(fv[:, None, :], (FROWS, LANES, LANES))
    grep = grep.reshape(ROW_TILE, LANES)
    sub = lax.broadcasted_iota(jnp.int32, (ROW_TILE, LANES), 0)
    lane = lax.broadcasted_iota(jnp.int32, (ROW_TILE, LANES), 1)
    sel = jnp.where(lane == (sub & (LANES - 1)), grep, 0.0)
    colflag = jnp.sum(sel, axis=-1, keepdims=True)

    mu = jnp.mean(xv, axis=-1, keepdims=True)
    xc = xv - mu
    var = jnp.mean(xc * xc, axis=-1, keepdims=True)
    ln = xc * lax.rsqrt(var + LN_EPS) * g_ref[...] + bt_ref[...]
    o_ref[0] = xv + ln * colflag


def kernel(x, seq_idx, seq_mask, node_type, gamma, beta):
    del node_type
    idx_flat = seq_idx.astype(jnp.int32).reshape(-1)
    msk_flat = seq_mask.astype(jnp.int32).reshape(-1)

    flags = _build_flag_kernel()(idx_flat, msk_flat)

    out = pl.pallas_call(
        _ln_body,
        grid=(B, N // ROW_TILE),
        in_specs=[
            pl.BlockSpec((1, ROW_TILE, C), lambda b, j: (b, j, 0)),
            pl.BlockSpec((1, FROWS, LANES), lambda b, j: (b, j, 0)),
            pl.BlockSpec((1, C), lambda b, j: (0, 0)),
            pl.BlockSpec((1, C), lambda b, j: (0, 0)),
        ],
        out_specs=pl.BlockSpec((1, ROW_TILE, C), lambda b, j: (b, j, 0)),
        out_shape=jax.ShapeDtypeStruct((B, N, C), jnp.float32),
    )(x, flags, gamma.reshape(1, C), beta.reshape(1, C))
    return out


# speedup vs baseline: 8.8946x; 1.0853x over previous
import functools

import jax
import jax.numpy as jnp
from jax import lax
from jax.experimental import pallas as pl
from jax.experimental.pallas import tpu as pltpu
from jax.experimental.pallas import tpu_sc as plsc

B, N, C, S = 8, 16384, 128, 8192
LN_EPS = 1e-5
LANES = 128
NROW = N // LANES


@functools.cache
def _build_flag_kernel():
    info = plsc.get_sparse_core_info()
    NC, NS, L = info.num_cores, info.num_subcores, info.num_lanes
    NW = NC * NS
    WPB = NW // B
    CHUNK = N // WPB
    CROWS = CHUNK // LANES

    mesh = plsc.VectorSubcoreMesh(core_axis_name="c", subcore_axis_name="s")

    @functools.partial(
        pl.kernel,
        mesh=mesh,
        out_type=jax.ShapeDtypeStruct((B, NROW, LANES), jnp.float32),
        scratch_types=[
            pltpu.VMEM((S,), jnp.int32),
            pltpu.VMEM((S,), jnp.int32),
            pltpu.VMEM((CROWS, LANES), jnp.float32),
        ],
        compiler_params=pltpu.CompilerParams(needs_layout_passes=False),
    )
    def flag_kernel(idx_hbm, msk_hbm, flags_hbm, idx_v, msk_v, flg_v):
        wid = lax.axis_index("s") * NC + lax.axis_index("c")
        b = wid // WPB
        base = (wid % WPB) * CHUNK

        zeros16 = jnp.zeros((L,), jnp.float32)

        def zero_body(i, carry):
            flg_v[i // (LANES // L), pl.ds((i % (LANES // L)) * L, L)] = zeros16
            return carry

        lax.fori_loop(0, CHUNK // L, zero_body, 0)

        pltpu.sync_copy(idx_hbm.at[pl.ds(b * S, S)], idx_v)
        pltpu.sync_copy(msk_hbm.at[pl.ds(b * S, S)], msk_v)

        ones16 = jnp.ones((L,), jnp.float32)

        def scatter_body(i, carry):
            ii = idx_v[pl.ds(i * L, L)]
            mm = msk_v[pl.ds(i * L, L)]
            ii = jnp.clip(ii, 0, N - 1)
            rel = ii - base
            ok = (mm != 0) & (rel >= 0) & (rel < CHUNK)
            rel = jnp.clip(rel, 0, CHUNK - 1)
            plsc.store_scatter(
                flg_v,
                [lax.shift_right_logical(rel, 7), rel & (LANES - 1)],
                ones16,
                mask=ok,
            )
            return carry

        lax.fori_loop(0, S // L, scatter_body, 0)

        pltpu.sync_copy(flg_v, flags_hbm.at[b, pl.ds((wid % WPB) * CROWS, CROWS)])

    return flag_kernel


ROW_TILE = 8192
FROWS = ROW_TILE // LANES


def _ln_body(x_ref, f_ref, g_ref, bt_ref, o_ref):
    xv = x_ref[0]
    fv = f_ref[0]
    grep = jnp.broadcast_to(fv[:, None, :], (FROWS, LANES, LANES))
    grep = grep.reshape(ROW_TILE, LANES)
    sub = lax.broadcasted_iota(jnp.int32, (ROW_TILE, LANES), 0)
    lane = lax.broadcasted_iota(jnp.int32, (ROW_TILE, LANES), 1)
    sel = jnp.where(lane == (sub & (LANES - 1)), grep, 0.0)
    colflag = jnp.sum(sel, axis=-1, keepdims=True)

    mu = jnp.mean(xv, axis=-1, keepdims=True)
    xc = xv - mu
    var = jnp.mean(xc * xc, axis=-1, keepdims=True)
    ln = xc * lax.rsqrt(var + LN_EPS) * g_ref[...] + bt_ref[...]
    o_ref[0] = xv + ln * colflag


def kernel(x, seq_idx, seq_mask, node_type, gamma, beta):
    del node_type
    idx_flat = seq_idx.astype(jnp.int32).reshape(-1)
    msk_flat = seq_mask.astype(jnp.int32).reshape(-1)

    flags = _build_flag_kernel()(idx_flat, msk_flat)

    out = pl.pallas_call(
        _ln_body,
        grid=(B, N // ROW_TILE),
        in_specs=[
            pl.BlockSpec((1, ROW_TILE, C), lambda b, j: (b, j, 0)),
            pl.BlockSpec((1, FROWS, LANES), lambda b, j: (b, j, 0)),
            pl.BlockSpec((1, C), lambda b, j: (0, 0)),
            pl.BlockSpec((1, C), lambda b, j: (0, 0)),
        ],
        out_specs=pl.BlockSpec((1, ROW_TILE, C), lambda b, j: (b, j, 0)),
        out_shape=jax.ShapeDtypeStruct((B, N, C), jnp.float32),
    )(x, flags, gamma.reshape(1, C), beta.reshape(1, C))
    return out
